# baseline (speedup 1.0000x reference)
import functools

import jax
import jax.numpy as jnp
from jax import lax
from jax.experimental import pallas as pl
from jax.experimental.pallas import tpu as pltpu
from jax.experimental.pallas import tpu_sc as plsc

N = 10000
E = 320000
D_IN = 128
H = 64
W128 = 128

NC = 2
NS = 16
NW = NC * NS
EPW = E // NW
CH = 40
NCH = EPW // CH
NSB = 25
SB = NCH // NSB
NBUF = 8
NP = 10240
RPT = NP // NS
NRC = RPT // CH
DR = NP // W128

_mesh = plsc.VectorSubcoreMesh(
    core_axis_name="c", subcore_axis_name="s", num_cores=NC, num_subcores=NS)
_sc_params = pltpu.CompilerParams(use_tc_tiling_on_sc=False)


def _fill_const(ref, nrows, width, val):
    v = jnp.full((16,), val, jnp.float32)

    def fill(i, c):
        for u in range(width // 16):
            ref[i, pl.ds(u * 16, 16)] = v
        return c
    lax.fori_loop(0, nrows, fill, 0)


W16 = 16

@functools.partial(
    pl.kernel,
    out_type=jax.ShapeDtypeStruct((NC, NP, W16), jnp.float32),
    mesh=_mesh,
    compiler_params=_sc_params,
    scratch_types=[
        pltpu.VMEM((NCH, CH), jnp.int32),
        pltpu.VMEM((CH, W16), jnp.float32),
        pltpu.SemaphoreType.DMA,
        pltpu.VMEM_SHARED((NP, W16), jnp.float32),
    ],
)
def _deg_kernel(dst_hbm, out_hbm, idx_d, buf, sem, deg_sh):
    cid = lax.axis_index("c")
    sid = lax.axis_index("s")
    wid = cid * NS + sid
    pltpu.sync_copy(dst_hbm.at[wid], idx_d)

    base = sid * RPT

    def zfill(i, c):
        buf[i, pl.ds(0, 16)] = jnp.zeros((16,), jnp.float32)
        return c
    lax.fori_loop(0, CH, zfill, 0)
    for t in range(NRC):
        pltpu.async_copy(buf, deg_sh.at[pl.ds(base + t * CH, CH)], sem)
    for t in range(NRC):
        pltpu.make_async_copy(buf, deg_sh.at[pl.ds(base, CH)], sem).wait()

    def ofill(i, c):
        buf[i, pl.ds(0, 16)] = jnp.full((16,), 1.0, jnp.float32)
        return c
    lax.fori_loop(0, CH, ofill, 0)
    plsc.subcore_barrier()

    def fire(j, c):
        pltpu.async_copy(buf, deg_sh.at[idx_d.at[j]], sem, add=True)
        return c
    lax.fori_loop(0, NCH, fire, 0)

    def drain(j, c):
        pltpu.make_async_copy(buf, deg_sh.at[idx_d.at[0]], sem).wait()
        return c
    lax.fori_loop(0, NCH, drain, 0)
    plsc.subcore_barrier()

    for t in range(NRC):
        sl = pl.ds(base + t * CH, CH)
        pltpu.async_copy(deg_sh.at[sl], out_hbm.at[cid, sl], sem)
    for t in range(NRC):
        pltpu.make_async_copy(deg_sh.at[pl.ds(base, CH)], out_hbm.at[cid, pl.ds(base, CH)], sem).wait()


@functools.partial(
    pl.kernel,
    out_type=jax.ShapeDtypeStruct((NC, NP, H), jnp.float32),
    mesh=_mesh,
    compiler_params=_sc_params,
    scratch_types=[
        pltpu.VMEM((3, SB, CH), jnp.int32),
        pltpu.VMEM((3, SB, CH), jnp.int32),
        pltpu.VMEM((NBUF, CH, H), jnp.float32),
        pltpu.SemaphoreType.DMA,
        pltpu.SemaphoreType.DMA,
        pltpu.SemaphoreType.DMA,
        pltpu.VMEM_SHARED((NP, H), jnp.float32),
    ],
)
def _agg_kernel(hs_hbm, src_hbm, dst_hbm, out_hbm, idx_s, idx_d, rows, gsem, ssem, isem, acc_sh):
    cid = lax.axis_index("c")
    sid = lax.axis_index("s")
    wid = cid * NS + sid
    pltpu.sync_copy(src_hbm.at[wid, 0], idx_s.at[0])
    pltpu.sync_copy(dst_hbm.at[wid, 0], idx_d.at[0])

    base = sid * RPT
    _fill_const(rows.at[0], CH, H, 0.0)
    for t in range(NRC):
        pltpu.async_copy(rows.at[0], acc_sh.at[pl.ds(base + t * CH, CH)], ssem)
    for t in range(NRC):
        pltpu.make_async_copy(rows.at[0], acc_sh.at[pl.ds(base, CH)], ssem).wait()
    plsc.subcore_barrier()

    def outer(b, c):
        rb = lax.rem(b, 3)
        nrb = lax.rem(b + 1, 3)
        prb = lax.rem(b + 2, 3)

        @pl.when(b + 1 < NSB)
        def _():
            pltpu.async_copy(src_hbm.at[wid, b + 1], idx_s.at[nrb], isem)
            pltpu.async_copy(dst_hbm.at[wid, b + 1], idx_d.at[nrb], isem)

        def inner(k, c2):
            j = b * SB + k
            buf = lax.rem(j, NBUF)
            pbuf = lax.rem(j + NBUF - 1, NBUF)

            @pl.when(j >= NBUF)
            def _():
                pltpu.make_async_copy(
                    rows.at[buf], acc_sh.at[idx_d.at[0, 0]], ssem).wait()

            pltpu.async_copy(hs_hbm.at[idx_s.at[rb, k]], rows.at[buf], gsem)

            @pl.when(j >= 1)
            def _():
                pltpu.make_async_copy(
                    hs_hbm.at[idx_s.at[0, 0]], rows.at[pbuf], gsem).wait()
                kk = lax.select(k == 0, SB - 1, k - 1)
                kb = lax.select(k == 0, prb, rb)
                pltpu.async_copy(
                    rows.at[pbuf], acc_sh.at[idx_d.at[kb, kk]], ssem, add=True)
            return c2
        lax.fori_loop(0, SB, inner, 0)

        @pl.when(b + 1 < NSB)
        def _():
            pltpu.make_async_copy(src_hbm.at[wid, 0], idx_s.at[nrb], isem).wait()
            pltpu.make_async_copy(dst_hbm.at[wid, 0], idx_d.at[nrb], isem).wait()
        return c
    lax.fori_loop(0, NSB, outer, 0)

    pltpu.make_async_copy(
        hs_hbm.at[idx_s.at[0, 0]], rows.at[(NCH - 1) % NBUF], gsem).wait()
    pltpu.async_copy(
        rows.at[(NCH - 1) % NBUF],
        acc_sh.at[idx_d.at[(NSB - 1) % 3, SB - 1]], ssem, add=True)
    for _ in range(NBUF):
        pltpu.make_async_copy(rows.at[0], acc_sh.at[idx_d.at[0, 0]], ssem).wait()
    plsc.subcore_barrier()

    for t in range(NRC):
        sl = pl.ds(base + t * CH, CH)
        pltpu.async_copy(acc_sh.at[sl], out_hbm.at[cid, sl], gsem)
    for t in range(NRC):
        pltpu.make_async_copy(acc_sh.at[pl.ds(base, CH)], out_hbm.at[cid, pl.ds(base, CH)], gsem).wait()


def _pre_body(x_ref, w_ref, d_ref, h_ref, hs_ref, dv_ref):
    deg = d_ref[0, :N, 0:1] + d_ref[1, :N, 0:1] + 1.0
    dinv = lax.rsqrt(deg)
    h = jnp.dot(x_ref[...], w_ref[...], preferred_element_type=jnp.float32)
    h_ref[...] = h
    hs_ref[...] = h * dinv
    dv_ref[...] = dinv


_pre_call = pl.pallas_call(
    _pre_body,
    out_shape=[
        jax.ShapeDtypeStruct((N, H), jnp.float32),
        jax.ShapeDtypeStruct((N, H), jnp.float32),
        jax.ShapeDtypeStruct((N, 1), jnp.float32),
    ],
)


def _layer_body(a_ref, hp_ref, dv_ref, b_ref, w_ref, h_ref, hs_ref):
    dv = dv_ref[...]
    agg = a_ref[0, :N, :] + a_ref[1, :N, :]
    x = dv * agg + dv * dv * hp_ref[...] + b_ref[...]
    x = jnp.maximum(x, 0.0)
    h = jnp.dot(x, w_ref[...], preferred_element_type=jnp.float32)
    h_ref[...] = h
    hs_ref[...] = h * dv


_layer_call = pl.pallas_call(
    _layer_body,
    out_shape=[
        jax.ShapeDtypeStruct((N, H), jnp.float32),
        jax.ShapeDtypeStruct((N, H), jnp.float32),
    ],
)


def _final_body(a_ref, hp_ref, dv_ref, b_ref, wp1_ref, wp2_ref, bp_ref, o_ref):
    dv = dv_ref[...]
    agg = a_ref[0, :N, :] + a_ref[1, :N, :]
    x = dv * agg + dv * dv * hp_ref[...] + b_ref[...]
    mean = jnp.sum(x, axis=0, keepdims=True) * (1.0 / N)
    mx = jnp.max(x, axis=0, keepdims=True)
    o_ref[...] = (jnp.dot(mean, wp1_ref[...], preferred_element_type=jnp.float32)
                  + jnp.dot(mx, wp2_ref[...], preferred_element_type=jnp.float32)
                  + bp_ref[...])


_final_call = pl.pallas_call(
    _final_body,
    out_shape=jax.ShapeDtypeStruct((1, H), jnp.float32),
)


def kernel(node_features, edge_index, W1, b1, W2, b2, W3, b3, Wp, bp):
    ei = edge_index.astype(jnp.int32)
    dst_r = ei[1].reshape(NW, NCH, CH)
    src4 = ei[0].reshape(NW, NSB, SB, CH)
    dst4 = ei[1].reshape(NW, NSB, SB, CH)

    deg2 = _deg_kernel(dst_r)
    h1, hs1, dinv = _pre_call(node_features, W1, deg2)

    acc = _agg_kernel(hs1, src4, dst4)
    h2, hs2 = _layer_call(acc, h1, dinv, b1.reshape(1, H), W2)
    acc = _agg_kernel(hs2, src4, dst4)
    h3, hs3 = _layer_call(acc, h2, dinv, b2.reshape(1, H), W3)
    acc = _agg_kernel(hs3, src4, dst4)

    return _final_call(acc, h3, dinv, b3.reshape(1, H),
                       Wp[:H], Wp[H:], bp.reshape(1, -1))


# speedup vs baseline: 1.2862x; 1.2862x over previous
import functools

import jax
import jax.numpy as jnp
from jax import lax
from jax.experimental import pallas as pl
from jax.experimental.pallas import tpu as pltpu
from jax.experimental.pallas import tpu_sc as plsc

N = 10000
E = 320000
D_IN = 128
H = 64
W128 = 128

NC = 2
NS = 16
NW = NC * NS
EPW = E // NW
CH = 80
NCH = EPW // CH
NSB = 25
SB = NCH // NSB
NBUF = 8
NP = 10240
RPT = NP // NS
NRC = RPT // CH
DR = NP // W128

_mesh = plsc.VectorSubcoreMesh(
    core_axis_name="c", subcore_axis_name="s", num_cores=NC, num_subcores=NS)
_sc_params = pltpu.CompilerParams(use_tc_tiling_on_sc=False)


def _fill_const(ref, nrows, width, val):
    v = jnp.full((16,), val, jnp.float32)

    def fill(i, c):
        for u in range(width // 16):
            ref[i, pl.ds(u * 16, 16)] = v
        return c
    lax.fori_loop(0, nrows, fill, 0)


W16 = 16

@functools.partial(
    pl.kernel,
    out_type=jax.ShapeDtypeStruct((NC, NP, W16), jnp.float32),
    mesh=_mesh,
    compiler_params=_sc_params,
    scratch_types=[
        pltpu.VMEM((NCH, CH), jnp.int32),
        pltpu.VMEM((CH, W16), jnp.float32),
        pltpu.SemaphoreType.DMA,
        pltpu.VMEM_SHARED((NP, W16), jnp.float32),
    ],
)
def _deg_kernel(dst_hbm, out_hbm, idx_d, buf, sem, deg_sh):
    cid = lax.axis_index("c")
    sid = lax.axis_index("s")
    wid = cid * NS + sid
    pltpu.sync_copy(dst_hbm.at[wid], idx_d)

    base = sid * RPT

    def zfill(i, c):
        buf[i, pl.ds(0, 16)] = jnp.zeros((16,), jnp.float32)
        return c
    lax.fori_loop(0, CH, zfill, 0)
    for t in range(NRC):
        pltpu.async_copy(buf, deg_sh.at[pl.ds(base + t * CH, CH)], sem)
    for t in range(NRC):
        pltpu.make_async_copy(buf, deg_sh.at[pl.ds(base, CH)], sem).wait()

    def ofill(i, c):
        buf[i, pl.ds(0, 16)] = jnp.full((16,), 1.0, jnp.float32)
        return c
    lax.fori_loop(0, CH, ofill, 0)
    plsc.subcore_barrier()

    def fire(j, c):
        pltpu.async_copy(buf, deg_sh.at[idx_d.at[j]], sem, add=True)
        return c
    lax.fori_loop(0, NCH, fire, 0)

    def drain(j, c):
        pltpu.make_async_copy(buf, deg_sh.at[idx_d.at[0]], sem).wait()
        return c
    lax.fori_loop(0, NCH, drain, 0)
    plsc.subcore_barrier()

    for t in range(NRC):
        sl = pl.ds(base + t * CH, CH)
        pltpu.async_copy(deg_sh.at[sl], out_hbm.at[cid, sl], sem)
    for t in range(NRC):
        pltpu.make_async_copy(deg_sh.at[pl.ds(base, CH)], out_hbm.at[cid, pl.ds(base, CH)], sem).wait()


@functools.partial(
    pl.kernel,
    out_type=jax.ShapeDtypeStruct((NC, NP, H), jnp.float32),
    mesh=_mesh,
    compiler_params=_sc_params,
    scratch_types=[
        pltpu.VMEM((3, SB, CH), jnp.int32),
        pltpu.VMEM((3, SB, CH), jnp.int32),
        pltpu.VMEM((NBUF, CH, H), jnp.float32),
        pltpu.SemaphoreType.DMA,
        pltpu.SemaphoreType.DMA,
        pltpu.SemaphoreType.DMA,
        pltpu.VMEM_SHARED((NP, H), jnp.float32),
    ],
)
def _agg_kernel(hs_hbm, src_hbm, dst_hbm, out_hbm, idx_s, idx_d, rows, gsem, ssem, isem, acc_sh):
    cid = lax.axis_index("c")
    sid = lax.axis_index("s")
    wid = cid * NS + sid
    pltpu.sync_copy(src_hbm.at[wid, 0], idx_s.at[0])
    pltpu.sync_copy(dst_hbm.at[wid, 0], idx_d.at[0])

    base = sid * RPT
    _fill_const(rows.at[0], CH, H, 0.0)
    for t in range(NRC):
        pltpu.async_copy(rows.at[0], acc_sh.at[pl.ds(base + t * CH, CH)], ssem)
    for t in range(NRC):
        pltpu.make_async_copy(rows.at[0], acc_sh.at[pl.ds(base, CH)], ssem).wait()
    plsc.subcore_barrier()

    def outer(b, c):
        rb = lax.rem(b, 3)
        nrb = lax.rem(b + 1, 3)
        prb = lax.rem(b + 2, 3)

        @pl.when(b + 1 < NSB)
        def _():
            pltpu.async_copy(src_hbm.at[wid, b + 1], idx_s.at[nrb], isem)
            pltpu.async_copy(dst_hbm.at[wid, b + 1], idx_d.at[nrb], isem)

        def inner(k, c2):
            j = b * SB + k
            buf = lax.rem(j, NBUF)
            pbuf = lax.rem(j + NBUF - 1, NBUF)

            @pl.when(j >= NBUF)
            def _():
                pltpu.make_async_copy(
                    rows.at[buf], acc_sh.at[idx_d.at[0, 0]], ssem).wait()

            pltpu.async_copy(hs_hbm.at[idx_s.at[rb, k]], rows.at[buf], gsem)

            @pl.when(j >= 1)
            def _():
                pltpu.make_async_copy(
                    hs_hbm.at[idx_s.at[0, 0]], rows.at[pbuf], gsem).wait()
                kk = lax.select(k == 0, SB - 1, k - 1)
                kb = lax.select(k == 0, prb, rb)
                pltpu.async_copy(
                    rows.at[pbuf], acc_sh.at[idx_d.at[kb, kk]], ssem, add=True)
            return c2
        lax.fori_loop(0, SB, inner, 0)

        @pl.when(b + 1 < NSB)
        def _():
            pltpu.make_async_copy(src_hbm.at[wid, 0], idx_s.at[nrb], isem).wait()
            pltpu.make_async_copy(dst_hbm.at[wid, 0], idx_d.at[nrb], isem).wait()
        return c
    lax.fori_loop(0, NSB, outer, 0)

    pltpu.make_async_copy(
        hs_hbm.at[idx_s.at[0, 0]], rows.at[(NCH - 1) % NBUF], gsem).wait()
    pltpu.async_copy(
        rows.at[(NCH - 1) % NBUF],
        acc_sh.at[idx_d.at[(NSB - 1) % 3, SB - 1]], ssem, add=True)
    for _ in range(NBUF):
        pltpu.make_async_copy(rows.at[0], acc_sh.at[idx_d.at[0, 0]], ssem).wait()
    plsc.subcore_barrier()

    for t in range(NRC):
        sl = pl.ds(base + t * CH, CH)
        pltpu.async_copy(acc_sh.at[sl], out_hbm.at[cid, sl], gsem)
    for t in range(NRC):
        pltpu.make_async_copy(acc_sh.at[pl.ds(base, CH)], out_hbm.at[cid, pl.ds(base, CH)], gsem).wait()


def _pre_body(x_ref, w_ref, d_ref, h_ref, hs_ref, dv_ref):
    deg = d_ref[0, :N, 0:1] + d_ref[1, :N, 0:1] + 1.0
    dinv = lax.rsqrt(deg)
    h = jnp.dot(x_ref[...], w_ref[...], preferred_element_type=jnp.float32)
    h_ref[...] = h
    hs_ref[...] = h * dinv
    dv_ref[...] = dinv


_pre_call = pl.pallas_call(
    _pre_body,
    out_shape=[
        jax.ShapeDtypeStruct((N, H), jnp.float32),
        jax.ShapeDtypeStruct((N, H), jnp.float32),
        jax.ShapeDtypeStruct((N, 1), jnp.float32),
    ],
)


def _layer_body(a_ref, hp_ref, dv_ref, b_ref, w_ref, h_ref, hs_ref):
    dv = dv_ref[...]
    agg = a_ref[0, :N, :] + a_ref[1, :N, :]
    x = dv * agg + dv * dv * hp_ref[...] + b_ref[...]
    x = jnp.maximum(x, 0.0)
    h = jnp.dot(x, w_ref[...], preferred_element_type=jnp.float32)
    h_ref[...] = h
    hs_ref[...] = h * dv


_layer_call = pl.pallas_call(
    _layer_body,
    out_shape=[
        jax.ShapeDtypeStruct((N, H), jnp.float32),
        jax.ShapeDtypeStruct((N, H), jnp.float32),
    ],
)


def _final_body(a_ref, hp_ref, dv_ref, b_ref, wp1_ref, wp2_ref, bp_ref, o_ref):
    dv = dv_ref[...]
    agg = a_ref[0, :N, :] + a_ref[1, :N, :]
    x = dv * agg + dv * dv * hp_ref[...] + b_ref[...]
    mean = jnp.sum(x, axis=0, keepdims=True) * (1.0 / N)
    mx = jnp.max(x, axis=0, keepdims=True)
    o_ref[...] = (jnp.dot(mean, wp1_ref[...], preferred_element_type=jnp.float32)
                  + jnp.dot(mx, wp2_ref[...], preferred_element_type=jnp.float32)
                  + bp_ref[...])


_final_call = pl.pallas_call(
    _final_body,
    out_shape=jax.ShapeDtypeStruct((1, H), jnp.float32),
)


def kernel(node_features, edge_index, W1, b1, W2, b2, W3, b3, Wp, bp):
    ei = edge_index.astype(jnp.int32)
    dst_r = ei[1].reshape(NW, NCH, CH)
    src4 = ei[0].reshape(NW, NSB, SB, CH)
    dst4 = ei[1].reshape(NW, NSB, SB, CH)

    deg2 = _deg_kernel(dst_r)
    h1, hs1, dinv = _pre_call(node_features, W1, deg2)

    acc = _agg_kernel(hs1, src4, dst4)
    h2, hs2 = _layer_call(acc, h1, dinv, b1.reshape(1, H), W2)
    acc = _agg_kernel(hs2, src4, dst4)
    h3, hs3 = _layer_call(acc, h2, dinv, b2.reshape(1, H), W3)
    acc = _agg_kernel(hs3, src4, dst4)

    return _final_call(acc, h3, dinv, b3.reshape(1, H),
                       Wp[:H], Wp[H:], bp.reshape(1, -1))
